# Initial kernel scaffold; baseline (speedup 1.0000x reference)
import jax
import jax.numpy as jnp
from jax.experimental import pallas as pl


def kernel(attr, attr_inds, attr_tf, attr_feat, attr_lens, attr_lens_user, attr_lens_item, user_ids, item_ids, pos_targets, pos_lens, neg_targets, neg_lens, attr_table, user_table, item_table, out_table):
    raise NotImplementedError("write your pallas kernel here")


# speedup vs baseline: 1.6838x; 1.6838x over previous
import functools

import jax
import jax.numpy as jnp
from jax import lax
from jax.experimental import pallas as pl
from jax.experimental.pallas import tpu as pltpu
from jax.experimental.pallas import tpu_sc as plsc

B = 4096
LA = 50
LAP = 56
LP = 20
LN = 200
LT = 224
D = 64
D3 = 192
NC = 2
NS = 16
NW = NC * NS
BPW = B // NW
CH = 8
NCH = BPW // CH


def _f32(x):
    return x.astype(jnp.float32)


def _sc_body(attr_hbm, tgt_hbm, uid_hbm, iid_hbm, scal_hbm,
             attr_tab, user_tab, item_tab, out_tab,
             out_logits, out_mask, out_ntg,
             s_aidx, s_tidx, s_uidx, s_iidx, s_scal,
             s_urows, s_irows, s_arows, s_trows,
             s_logits, s_mask, s_ntg,
             sem_a, sem_b, sem_c):
    wid = lax.axis_index("s") * NC + lax.axis_index("c")

    iota = lax.iota(jnp.int32, 16)

    def chunk_body(ch, _):
        base = wid * BPW + ch * CH
        pltpu.sync_copy(attr_hbm.at[pl.ds(base, CH)], s_aidx)
        pltpu.sync_copy(tgt_hbm.at[pl.ds(base, CH)], s_tidx)
        pltpu.sync_copy(uid_hbm.at[pl.ds(base, CH)], s_uidx)
        pltpu.sync_copy(iid_hbm.at[pl.ds(base, CH)], s_iidx)
        pltpu.sync_copy(scal_hbm.at[pl.ds(base, CH)], s_scal)
        pltpu.async_copy(user_tab.at[s_uidx], s_urows, sem_a).wait()
        pltpu.async_copy(item_tab.at[s_iidx], s_irows, sem_a).wait()

        def row_body(j, _):
            cp_attr = pltpu.async_copy(attr_tab.at[s_aidx.at[j]], s_arows,
                                       sem_b)
            cp_t0 = pltpu.async_copy(out_tab.at[s_tidx.at[j, 0]],
                                     s_trows.at[pl.ds(0, 112)], sem_c)
            cp_t1 = pltpu.async_copy(out_tab.at[s_tidx.at[j, 1]],
                                     s_trows.at[pl.ds(112, 112)], sem_c)

            svec = s_scal[j, :]
            alen = jnp.minimum(svec[0], LA)
            plen = svec[1]
            nlen = svec[2]

            cp_attr.wait()

            def attr_body(i, accs):
                return tuple(
                    accs[k] + s_arows[i, pl.ds(k * 16, 16)] for k in range(4))

            accs = lax.fori_loop(0, alen, attr_body,
                                 tuple(jnp.zeros((16,), jnp.float32)
                                       for _ in range(4)))
            inv_v = jnp.full((16,), 1.0, jnp.float32) / jnp.full(
                (16,), _f32(alen), jnp.float32)
            uio = ([s_urows[j, pl.ds(k * 16, 16)] for k in range(4)]
                   + [accs[k] * inv_v for k in range(4)]
                   + [s_irows[j, pl.ds(k * 16, 16)] for k in range(4)])

            plen_v = jnp.full((16,), plen, jnp.int32)
            nlen_v = jnp.full((16,), nlen, jnp.int32)
            zero_v = jnp.zeros((16,), jnp.int32)
            for g in range(LT // 16):
                p = iota + (g * 16)
                is_pos = p < LP
                m = jnp.where(is_pos, p < plen_v, (p - LP) < nlen_v)
                mi = m.astype(jnp.int32)
                s_mask[j, pl.ds(g * 16, 16)] = mi
                s_ntg[j, pl.ds(g * 16, 16)] = jnp.where(is_pos, mi, zero_v)

            cp_t0.wait()
            cp_t1.wait()

            def dot_body(it, _):
                r0 = it * 16
                res = jnp.zeros((16,), jnp.float32)
                for u in range(16):
                    r = r0 + u
                    prods = [s_trows[r, pl.ds(k * 16, 16)] * uio[k]
                             for k in range(12)]
                    while len(prods) > 1:
                        prods = [prods[i] + prods[i + 1]
                                 for i in range(0, len(prods) - 1, 2)] + (
                            [prods[-1]] if len(prods) % 2 else [])
                    s_v = jnp.full((16,), jnp.sum(prods[0]), jnp.float32)
                    res = jnp.where(iota == u, s_v, res)
                s_logits[j, pl.ds(r0, 16)] = res
                return 0

            lax.fori_loop(0, LT // 16, dot_body, 0)
            return 0

        lax.fori_loop(0, CH, row_body, 0)

        pltpu.sync_copy(s_logits, out_logits.at[pl.ds(base, CH)])
        pltpu.sync_copy(s_mask, out_mask.at[pl.ds(base, CH)])
        pltpu.sync_copy(s_ntg, out_ntg.at[pl.ds(base, CH)])
        return 0

    lax.fori_loop(0, NCH, chunk_body, 0)


@jax.jit
def _run(attr_p, tgt3, user_ids, item_ids, scal4,
         attr_table, user_table, item_table, out_table):
    mesh = plsc.VectorSubcoreMesh(core_axis_name="c", subcore_axis_name="s",
                                  num_cores=NC, num_subcores=NS)
    f = pl.kernel(
        _sc_body,
        out_type=(
            jax.ShapeDtypeStruct((B, LT), jnp.float32),
            jax.ShapeDtypeStruct((B, LT), jnp.int32),
            jax.ShapeDtypeStruct((B, LT), jnp.int32),
        ),
        mesh=mesh,
        scratch_types=[
            pltpu.VMEM((CH, LAP), jnp.int32),
            pltpu.VMEM((CH, 2, 112), jnp.int32),
            pltpu.VMEM((CH,), jnp.int32),
            pltpu.VMEM((CH,), jnp.int32),
            pltpu.VMEM((CH, 16), jnp.int32),
            pltpu.VMEM((CH, D), jnp.float32),
            pltpu.VMEM((CH, D), jnp.float32),
            pltpu.VMEM((LAP, D), jnp.float32),
            pltpu.VMEM((LT, D3), jnp.float32),
            pltpu.VMEM((CH, LT), jnp.float32),
            pltpu.VMEM((CH, LT), jnp.int32),
            pltpu.VMEM((CH, LT), jnp.int32),
            pltpu.SemaphoreType.DMA,
            pltpu.SemaphoreType.DMA,
            pltpu.SemaphoreType.DMA,
        ],
        compiler_params=pltpu.CompilerParams(use_tc_tiling_on_sc=False,
                                             needs_layout_passes=False),
    )
    return f(attr_p, tgt3, user_ids, item_ids, scal4,
             attr_table, user_table, item_table, out_table)


def kernel(attr, attr_inds, attr_tf, attr_feat, attr_lens, attr_lens_user,
           attr_lens_item, user_ids, item_ids, pos_targets, pos_lens,
           neg_targets, neg_lens, attr_table, user_table, item_table,
           out_table):
    attr_p = jnp.pad(attr, ((0, 0), (0, LAP - LA)))
    tgt = jnp.concatenate(
        [pos_targets, neg_targets,
         jnp.zeros((B, LT - LP - LN), jnp.int32)], axis=1)
    tgt3 = tgt.reshape(B, 2, 112)
    scal4 = jnp.concatenate(
        [attr_lens[:, None], pos_lens[:, None], neg_lens[:, None],
         jnp.zeros((B, 13), jnp.int32)], axis=1)
    logits_p, mask_i, ntg_i = _run(
        attr_p, tgt3, user_ids, item_ids, scal4,
        attr_table, user_table, item_table, out_table)
    logits = logits_p[:, :LP + LN]
    mask = mask_i[:, :LP + LN].astype(bool)
    new_targets = ntg_i[:, :LP + LN]
    return (logits, mask, new_targets)
